# baseline (speedup 1.0000x reference)
import functools

import jax
import jax.numpy as jnp
from jax import lax
from jax.experimental import pallas as pl
from jax.experimental.pallas import tpu as pltpu
from jax.experimental.pallas import tpu_sc as plsc

N = 10000
P = 320000
F = 128
NBF = 32

NC = 2
NS = 16
LANES = 16

EDGES_PER_TILE = P // (NC * NS)
EB = 40
NBLK = EDGES_PER_TILE // EB
CHUNK = 50
NCHUNK = NBLK // CHUNK
ROWS_PER_TILE = 624


def _swish(v):
    return v * jax.nn.sigmoid(v)


def _node_mlps_body(x_ref, *refs):
    w_refs = refs[:24]
    out_refs = refs[24:]
    x = x_ref[...]
    for br in range(4):
        w1t, b1, w2t, b2, wot, bo = w_refs[br * 6:(br + 1) * 6]
        y = _swish(x)
        y = jnp.dot(y, w1t[...], preferred_element_type=jnp.float32) + b1[...][None, :]
        y = _swish(y)
        y = jnp.dot(y, w2t[...], preferred_element_type=jnp.float32) + b2[...][None, :]
        h = x + y
        h = _swish(h)
        out_refs[br][...] = jnp.dot(h, wot[...], preferred_element_type=jnp.float32) + bo[...][None, :]


def _node_mlps(x, branch_params):
    blk = 1000
    grid = N // blk
    flat_w = []
    for bp in branch_params:
        blkp = bp["blocks"][0]
        flat_w += [
            blkp["lin1"]["w"].T, blkp["lin1"]["b"],
            blkp["lin2"]["w"].T, blkp["lin2"]["b"],
            bp["out"]["w"].T, bp["out"]["b"],
        ]
    w_specs = []
    for w in flat_w:
        nd = w.ndim
        w_specs.append(pl.BlockSpec(w.shape, (lambda i, nd=nd: (0,) * nd)))
    return pl.pallas_call(
        _node_mlps_body,
        grid=(grid,),
        in_specs=[pl.BlockSpec((blk, F), lambda i: (i, 0))] + w_specs,
        out_specs=[pl.BlockSpec((blk, F), lambda i: (i, 0))] * 4,
        out_shape=[jax.ShapeDtypeStruct((N, F), jnp.float32)] * 4,
    )(x, *flat_w)


def _radial_plane_body(geo_k, rbf_ref, geo_ref, w_ref, out_ref):
    g = jnp.dot(rbf_ref[...], w_ref[...], preferred_element_type=jnp.float32)
    if geo_k is not None:
        g = g * geo_ref[...][:, geo_k:geo_k + 1]
    out_ref[...] = g


def _radial_plane(rbf, geo, w_t, geo_k):
    blk = 2000
    grid = P // blk
    gdim = geo.shape[1]
    return pl.pallas_call(
        functools.partial(_radial_plane_body, geo_k),
        grid=(grid,),
        in_specs=[pl.BlockSpec((blk, NBF), lambda i: (i, 0)),
                  pl.BlockSpec((blk, gdim), lambda i: (i, 0)),
                  pl.BlockSpec((NBF, F), lambda i: (0, 0))],
        out_specs=pl.BlockSpec((blk, F), lambda i: (i, 0)),
        out_shape=jax.ShapeDtypeStruct((P, F), jnp.float32),
    )(rbf, geo, w_t)


def _sc_pass_body(table, coeff, idx_i, idx_j, zeros, out,
                  acc, idxi_c, idxj_c, gath_v, coef_v, msg_v,
                  semi0, semi1, sems0, sems1):
    sems_in = (semi0, semi1)
    sems_sc = (sems0, sems1)

    c = lax.axis_index("c")
    s = lax.axis_index("s")
    w = c * NS + s
    base = w * EDGES_PER_TILE

    @pl.when(s == 0)
    def _init():
        pltpu.sync_copy(zeros, acc)

    plsc.subcore_barrier()

    def chunk(ch, carry):
        cbase = base + ch * CHUNK * EB
        pltpu.sync_copy(idx_i.at[w, ch], idxi_c)
        pltpu.sync_copy(idx_j.at[w, ch], idxj_c)

        def issue(b, slot):
            sem = sems_in[slot]
            pltpu.async_copy(table.at[idxj_c.at[b]], gath_v.at[slot], sem)
            pltpu.async_copy(coeff.at[pl.ds(cbase + b * EB, EB), :], coef_v.at[slot], sem)

        def wait_inputs(b, slot):
            sem = sems_in[slot]
            pltpu.make_async_copy(table.at[idxj_c.at[b]], gath_v.at[slot], sem).wait()
            pltpu.make_async_copy(coeff.at[pl.ds(cbase + b * EB, EB), :], coef_v.at[slot], sem).wait()

        def compute(slot):
            def row(r, carry2):
                for k in range(F // LANES):
                    sl = pl.ds(k * LANES, LANES)
                    msg_v[slot, r, sl] = coef_v[slot, r, sl] * gath_v[slot, r, sl]
                return carry2

            lax.fori_loop(0, EB, row, 0, unroll=False)

        def start_scatter(b, slot):
            pltpu.async_copy(msg_v.at[slot], acc.at[idxi_c.at[b]], sems_sc[slot], add=True)

        def wait_scatter(b, slot):
            pltpu.make_async_copy(msg_v.at[slot], acc.at[idxi_c.at[b]], sems_sc[slot]).wait()

        issue(0, 0)
        issue(1, 1)
        wait_inputs(0, 0)
        compute(0)
        start_scatter(0, 0)
        issue(2, 0)
        wait_inputs(1, 1)
        compute(1)
        start_scatter(1, 1)
        issue(3, 1)

        def pair(bb, carry2):
            b0 = bb * 2
            wait_inputs(b0, 0)
            wait_scatter(b0 - 2, 0)
            compute(0)
            start_scatter(b0, 0)
            issue(b0 + 2, 0)
            wait_inputs(b0 + 1, 1)
            wait_scatter(b0 - 1, 1)
            compute(1)
            start_scatter(b0 + 1, 1)
            issue(b0 + 3, 1)
            return carry2

        lax.fori_loop(1, CHUNK // 2 - 1, pair, 0, unroll=False)

        wait_inputs(CHUNK - 2, 0)
        wait_scatter(CHUNK - 4, 0)
        compute(0)
        start_scatter(CHUNK - 2, 0)
        wait_inputs(CHUNK - 1, 1)
        wait_scatter(CHUNK - 3, 1)
        compute(1)
        start_scatter(CHUNK - 1, 1)
        wait_scatter(CHUNK - 2, 0)
        wait_scatter(CHUNK - 1, 1)
        return carry

    lax.fori_loop(0, NCHUNK, chunk, 0, unroll=False)

    plsc.subcore_barrier()
    pltpu.sync_copy(acc.at[pl.ds(s * ROWS_PER_TILE, ROWS_PER_TILE), :],
                    out.at[c, pl.ds(s * ROWS_PER_TILE, ROWS_PER_TILE), :])

    @pl.when(s == NS - 1)
    def _tail():
        pltpu.sync_copy(acc.at[pl.ds(NS * ROWS_PER_TILE, N - NS * ROWS_PER_TILE), :],
                        out.at[c, pl.ds(NS * ROWS_PER_TILE, N - NS * ROWS_PER_TILE), :])


def _make_sc_pass():
    mesh = plsc.VectorSubcoreMesh(
        core_axis_name="c", subcore_axis_name="s", num_cores=NC, num_subcores=NS)
    scratch = [
        pltpu.VMEM_SHARED((N, F), jnp.float32),
        pltpu.VMEM((CHUNK, EB), jnp.int32),
        pltpu.VMEM((CHUNK, EB), jnp.int32),
        pltpu.VMEM((2, EB, F), jnp.float32),
        pltpu.VMEM((2, EB, F), jnp.float32),
        pltpu.VMEM((2, EB, F), jnp.float32),
        pltpu.SemaphoreType.DMA,
        pltpu.SemaphoreType.DMA,
        pltpu.SemaphoreType.DMA,
        pltpu.SemaphoreType.DMA,
    ]
    return pl.kernel(
        _sc_pass_body,
        out_type=jax.ShapeDtypeStruct((NC, N, F), jnp.float32),
        mesh=mesh,
        scratch_types=scratch,
    )


def _combine_body(*refs):
    xx_ref = refs[0]
    planes = refs[1:10]
    ppt, pdt, w1t, b1, w2t, b2, wot, bo = refs[10:18]
    out_ref = refs[18]

    s2 = planes[0][...]
    tot = xx_ref[...] + s2[0] + s2[1]
    for k in range(3):
        pk2 = planes[1 + k][...]
        pk = pk2[0] + pk2[1]
        ab = jnp.dot(pk, ppt[...], preferred_element_type=jnp.float32)
        tot = tot + ab[:, :F] * ab[:, F:]
    for k in range(5):
        dk2 = planes[4 + k][...]
        dk = dk2[0] + dk2[1]
        ab = jnp.dot(dk, pdt[...], preferred_element_type=jnp.float32)
        tot = tot + ab[:, :F] * ab[:, F:]

    y = _swish(tot)
    y = jnp.dot(y, w1t[...], preferred_element_type=jnp.float32) + b1[...][None, :]
    y = _swish(y)
    y = jnp.dot(y, w2t[...], preferred_element_type=jnp.float32) + b2[...][None, :]
    h = tot + y
    h = _swish(h)
    out_ref[...] = jnp.dot(h, wot[...], preferred_element_type=jnp.float32) + bo[...][None, :]


def _combine(xx, planes, proj_p_t, proj_d_t, res_params):
    blk = 1000
    grid = N // blk
    blkp = res_params["blocks"][0]
    ws = [proj_p_t, proj_d_t,
          blkp["lin1"]["w"].T, blkp["lin1"]["b"],
          blkp["lin2"]["w"].T, blkp["lin2"]["b"],
          res_params["out"]["w"].T, res_params["out"]["b"]]
    w_specs = []
    for w in ws:
        nd = w.ndim
        w_specs.append(pl.BlockSpec(w.shape, (lambda i, nd=nd: (0,) * nd)))
    return pl.pallas_call(
        _combine_body,
        grid=(grid,),
        in_specs=[pl.BlockSpec((blk, F), lambda i: (i, 0))]
        + [pl.BlockSpec((NC, blk, F), lambda i: (0, i, 0))] * 9 + w_specs,
        out_specs=pl.BlockSpec((blk, F), lambda i: (i, 0)),
        out_shape=jax.ShapeDtypeStruct((N, F), jnp.float32),
    )(xx, *planes, *ws)


def kernel(x, rbf, pij, dij, idx_i, idx_j, params):
    xx, hs, hp, hd = _node_mlps(
        x, [params["resblock_x"], params["resblock_s"],
            params["resblock_p"], params["resblock_d"]])

    ws_t = params["radial_s"].T
    wp_t = params["radial_p"].T
    wd_t = params["radial_d"].T
    plane_specs = [(pij, ws_t, None)] \
        + [(pij, wp_t, k) for k in range(3)] \
        + [(dij, wd_t, k) for k in range(5)]

    zeros = jnp.zeros((N, F), jnp.float32)
    idx_i = idx_i.reshape(NC * NS, NCHUNK, CHUNK, EB)
    idx_j = idx_j.reshape(NC * NS, NCHUNK, CHUNK, EB)

    sc_pass = _make_sc_pass()
    tables = [hs] + [hp] * 3 + [hd] * 5
    planes = []
    for i in range(9):
        geo, w_t, gk = plane_specs[i]
        coeff_i = _radial_plane(rbf, geo, w_t, gk)
        planes.append(sc_pass(tables[i], coeff_i, idx_i, idx_j, zeros))

    out = _combine(xx, planes, params["projection_p"].T, params["projection_d"].T,
                   params["resblock"])
    return out


# speedup vs baseline: 1.0837x; 1.0837x over previous
import functools

import jax
import jax.numpy as jnp
from jax import lax
from jax.experimental import pallas as pl
from jax.experimental.pallas import tpu as pltpu
from jax.experimental.pallas import tpu_sc as plsc

N = 10000
P = 320000
F = 128
NBF = 32

NC = 2
NS = 16
LANES = 16

EDGES_PER_TILE = P // (NC * NS)
EB = 80
NBLK = EDGES_PER_TILE // EB
CHUNK = 25
NCHUNK = NBLK // CHUNK
NBLKG = P // EB
ROWS_PER_TILE = 624


def _swish(v):
    return v * jax.nn.sigmoid(v)


def _node_mlps_body(x_ref, *refs):
    w_refs = refs[:24]
    out_refs = refs[24:]
    x = x_ref[...]
    for br in range(4):
        w1t, b1, w2t, b2, wot, bo = w_refs[br * 6:(br + 1) * 6]
        y = _swish(x)
        y = jnp.dot(y, w1t[...], preferred_element_type=jnp.float32) + b1[...][None, :]
        y = _swish(y)
        y = jnp.dot(y, w2t[...], preferred_element_type=jnp.float32) + b2[...][None, :]
        h = x + y
        h = _swish(h)
        h = jnp.dot(h, wot[...], preferred_element_type=jnp.float32) + bo[...][None, :]
        if br == 0:
            out_refs[0][...] = h
        else:
            out_refs[1][br - 1] = h


def _node_mlps(x, branch_params):
    blk = 1000
    grid = N // blk
    flat_w = []
    for bp in branch_params:
        blkp = bp["blocks"][0]
        flat_w += [
            blkp["lin1"]["w"].T, blkp["lin1"]["b"],
            blkp["lin2"]["w"].T, blkp["lin2"]["b"],
            bp["out"]["w"].T, bp["out"]["b"],
        ]
    w_specs = []
    for w in flat_w:
        nd = w.ndim
        w_specs.append(pl.BlockSpec(w.shape, (lambda i, nd=nd: (0,) * nd)))
    return pl.pallas_call(
        _node_mlps_body,
        grid=(grid,),
        in_specs=[pl.BlockSpec((blk, F), lambda i: (i, 0))] + w_specs,
        out_specs=[pl.BlockSpec((blk, F), lambda i: (i, 0)),
                   pl.BlockSpec((3, blk, F), lambda i: (0, i, 0))],
        out_shape=[jax.ShapeDtypeStruct((N, F), jnp.float32),
                   jax.ShapeDtypeStruct((3, N, F), jnp.float32)],
    )(x, *flat_w)


def _radial_body(rbf_ref, pij_ref, dij_ref, ws_ref, wp_ref, wd_ref, out_ref):
    r = rbf_ref[...]
    gs = jnp.dot(r, ws_ref[...], preferred_element_type=jnp.float32)
    gp = jnp.dot(r, wp_ref[...], preferred_element_type=jnp.float32)
    gd = jnp.dot(r, wd_ref[...], preferred_element_type=jnp.float32)
    out_ref[0] = gs
    for k in range(3):
        out_ref[1 + k] = gp * pij_ref[...][:, k:k + 1]
    for k in range(5):
        out_ref[4 + k] = gd * dij_ref[...][:, k:k + 1]


def _radial(rbf, pij, dij, ws_t, wp_t, wd_t):
    blk = 2000
    grid = P // blk
    wspec = pl.BlockSpec((NBF, F), lambda i: (0, 0))
    return pl.pallas_call(
        _radial_body,
        grid=(grid,),
        in_specs=[pl.BlockSpec((blk, NBF), lambda i: (i, 0)),
                  pl.BlockSpec((blk, 3), lambda i: (i, 0)),
                  pl.BlockSpec((blk, 5), lambda i: (i, 0)),
                  wspec, wspec, wspec],
        out_specs=pl.BlockSpec((9, blk, F), lambda i: (0, i, 0)),
        out_shape=jax.ShapeDtypeStruct((9, P, F), jnp.float32),
    )(rbf, pij, dij, ws_t, wp_t, wd_t)


def _sc_all_body(tables, coeffs, idx_i, idx_j, zeros, out,
                 acc, idxi_c, idxj_c, gath_v, coef_v,
                 semi0, semi1):
    sems_in = (semi0, semi1)

    c = lax.axis_index("c")
    s = lax.axis_index("s")
    w = c * NS + s
    base = w * EDGES_PER_TILE

    def phase(p, carry0):
        tsel = jnp.where(p == 0, 0, jnp.where(p <= 3, 1, 2))

        @pl.when(s == 0)
        def _init():
            pltpu.sync_copy(zeros, acc)

        plsc.subcore_barrier()

        def chunk(ch, carry):
            cbase = base + ch * CHUNK * EB
            pltpu.sync_copy(idx_i.at[w, ch], idxi_c)
            pltpu.sync_copy(idx_j.at[w, ch], idxj_c)

            def issue(b, slot):
                sem = sems_in[slot]
                pltpu.async_copy(tables.at[tsel].at[idxj_c.at[b]], gath_v.at[slot], sem)
                pltpu.async_copy(coeffs.at[p, pl.ds(cbase + b * EB, EB), :],
                                 coef_v.at[slot], sem)

            def consume(b, slot):
                sem = sems_in[slot]
                pltpu.make_async_copy(tables.at[0].at[idxj_c.at[b]], gath_v.at[slot], sem).wait()
                pltpu.make_async_copy(coeffs.at[p, pl.ds(cbase + b * EB, EB), :],
                                      coef_v.at[slot], sem).wait()

                def row(r, carry2):
                    for k in range(F // LANES):
                        sl = pl.ds(k * LANES, LANES)
                        gath_v[slot, r, sl] = coef_v[slot, r, sl] * gath_v[slot, r, sl]
                    return carry2

                lax.fori_loop(0, EB, row, 0, unroll=False)
                pltpu.sync_copy(gath_v.at[slot], acc.at[idxi_c.at[b]], add=True)

            issue(0, 0)

            def pair(bb, carry2):
                b0 = bb * 2
                issue(b0 + 1, 1)
                consume(b0, 0)
                issue(b0 + 2, 0)
                consume(b0 + 1, 1)
                return carry2

            lax.fori_loop(0, (CHUNK - 1) // 2, pair, 0, unroll=False)
            consume(CHUNK - 1, 0)
            return carry

        lax.fori_loop(0, NCHUNK, chunk, 0, unroll=False)

        plsc.subcore_barrier()
        pltpu.sync_copy(acc.at[pl.ds(s * ROWS_PER_TILE, ROWS_PER_TILE), :],
                        out.at[p, c, pl.ds(s * ROWS_PER_TILE, ROWS_PER_TILE), :])

        @pl.when(s == NS - 1)
        def _tail():
            pltpu.sync_copy(acc.at[pl.ds(NS * ROWS_PER_TILE, N - NS * ROWS_PER_TILE), :],
                            out.at[p, c, pl.ds(NS * ROWS_PER_TILE, N - NS * ROWS_PER_TILE), :])

        plsc.subcore_barrier()
        return carry0

    lax.fori_loop(0, 9, phase, 0, unroll=False)


def _make_sc_all():
    mesh = plsc.VectorSubcoreMesh(
        core_axis_name="c", subcore_axis_name="s", num_cores=NC, num_subcores=NS)
    scratch = [
        pltpu.VMEM_SHARED((N, F), jnp.float32),
        pltpu.VMEM((CHUNK, EB), jnp.int32),
        pltpu.VMEM((CHUNK, EB), jnp.int32),
        pltpu.VMEM((2, EB, F), jnp.float32),
        pltpu.VMEM((2, EB, F), jnp.float32),
        pltpu.SemaphoreType.DMA,
        pltpu.SemaphoreType.DMA,
    ]
    return pl.kernel(
        _sc_all_body,
        out_type=jax.ShapeDtypeStruct((9, NC, N, F), jnp.float32),
        mesh=mesh,
        scratch_types=scratch,
    )


def _combine_body(*refs):
    xx_ref = refs[0]
    planes_ref = refs[1]
    ppt, pdt, w1t, b1, w2t, b2, wot, bo = refs[2:10]
    out_ref = refs[10]

    tot = xx_ref[...] + planes_ref[0, 0] + planes_ref[0, 1]
    for k in range(3):
        pk = planes_ref[1 + k, 0] + planes_ref[1 + k, 1]
        ab = jnp.dot(pk, ppt[...], preferred_element_type=jnp.float32)
        tot = tot + ab[:, :F] * ab[:, F:]
    for k in range(5):
        dk = planes_ref[4 + k, 0] + planes_ref[4 + k, 1]
        ab = jnp.dot(dk, pdt[...], preferred_element_type=jnp.float32)
        tot = tot + ab[:, :F] * ab[:, F:]

    y = _swish(tot)
    y = jnp.dot(y, w1t[...], preferred_element_type=jnp.float32) + b1[...][None, :]
    y = _swish(y)
    y = jnp.dot(y, w2t[...], preferred_element_type=jnp.float32) + b2[...][None, :]
    h = tot + y
    h = _swish(h)
    out_ref[...] = jnp.dot(h, wot[...], preferred_element_type=jnp.float32) + bo[...][None, :]


def _combine(xx, planes, proj_p_t, proj_d_t, res_params):
    blk = 1000
    grid = N // blk
    blkp = res_params["blocks"][0]
    ws = [proj_p_t, proj_d_t,
          blkp["lin1"]["w"].T, blkp["lin1"]["b"],
          blkp["lin2"]["w"].T, blkp["lin2"]["b"],
          res_params["out"]["w"].T, res_params["out"]["b"]]
    w_specs = []
    for w in ws:
        nd = w.ndim
        w_specs.append(pl.BlockSpec(w.shape, (lambda i, nd=nd: (0,) * nd)))
    return pl.pallas_call(
        _combine_body,
        grid=(grid,),
        in_specs=[pl.BlockSpec((blk, F), lambda i: (i, 0)),
                  pl.BlockSpec((9, NC, blk, F), lambda i: (0, 0, i, 0))] + w_specs,
        out_specs=pl.BlockSpec((blk, F), lambda i: (i, 0)),
        out_shape=jax.ShapeDtypeStruct((N, F), jnp.float32),
    )(xx, planes, *ws)


def kernel(x, rbf, pij, dij, idx_i, idx_j, params):
    xx, tables = _node_mlps(
        x, [params["resblock_x"], params["resblock_s"],
            params["resblock_p"], params["resblock_d"]])

    coeffs = _radial(rbf, pij, dij,
                     params["radial_s"].T, params["radial_p"].T, params["radial_d"].T)

    zeros = jnp.zeros((N, F), jnp.float32)
    idx_i = idx_i.reshape(NC * NS, NCHUNK, CHUNK, EB)
    idx_j = idx_j.reshape(NC * NS, NCHUNK, CHUNK, EB)

    planes = _make_sc_all()(tables, coeffs, idx_i, idx_j, zeros)

    out = _combine(xx, planes, params["projection_p"].T, params["projection_d"].T,
                   params["resblock"])
    return out
